# Initial kernel scaffold; baseline (speedup 1.0000x reference)
import jax
import jax.numpy as jnp
from jax.experimental import pallas as pl


def kernel(x):
    raise NotImplementedError("write your pallas kernel here")


# speedup vs baseline: 12.4691x; 12.4691x over previous
import functools

import jax
import jax.numpy as jnp
from jax import lax
from jax.experimental import pallas as pl
from jax.experimental.pallas import tpu as pltpu
from jax.experimental.pallas import tpu_sc as plsc

_DROP_RATIO = 0.75


def _sc_gather(xf, flat_idx, BK, D):
    info = plsc.get_sparse_core_info()
    NC, NS = info.num_cores, info.num_subcores
    NW = NC * NS
    b_per_w = BK // NW
    CH = 128
    n_ch = b_per_w // CH

    mesh = plsc.VectorSubcoreMesh(core_axis_name="c", subcore_axis_name="s")

    @functools.partial(
        pl.kernel,
        mesh=mesh,
        out_type=jax.ShapeDtypeStruct((BK, D), jnp.float32),
        scratch_types=[
            pltpu.VMEM((CH,), jnp.int32),
            pltpu.VMEM((CH, D), jnp.float32),
            pltpu.SemaphoreType.DMA,
        ],
    )
    def gather_kernel(x_hbm, idx_hbm, out_hbm, idx_v, rows_v, sem):
        wid = lax.axis_index("s") * NC + lax.axis_index("c")
        base = wid * b_per_w
        for c in range(n_ch):
            off = base + c * CH
            pltpu.sync_copy(idx_hbm.at[pl.ds(off, CH)], idx_v)
            pltpu.async_copy(x_hbm.at[idx_v], rows_v, sem).wait()
            pltpu.sync_copy(rows_v, out_hbm.at[pl.ds(off, CH)])

    return gather_kernel(xf, flat_idx)


def kernel(x):
    B, N, D = x.shape
    keep_ratio = 1.0 - _DROP_RATIO
    num_keep = max(1, int(N * keep_ratio))
    step = N / num_keep
    base_indices = jnp.arange(num_keep, dtype=jnp.float32) * step
    offs_key = jax.random.key(42)
    offsets = jax.random.uniform(offs_key, (B, num_keep), dtype=jnp.float32) * (step * 0.5)
    indices = (base_indices[None, :] + offsets).astype(jnp.int32)
    indices = jnp.clip(indices, 0, N - 1)

    flat_idx = (indices + jnp.arange(B, dtype=jnp.int32)[:, None] * N).reshape(-1)
    xf = x.reshape(B * N, D)
    out = _sc_gather(xf, flat_idx, B * num_keep, D)
    return out.reshape(B, num_keep, D), indices
